# Initial kernel scaffold; baseline (speedup 1.0000x reference)
import jax
import jax.numpy as jnp
from jax.experimental import pallas as pl


def kernel(q_tensor, k_tensor, v_tensor):
    raise NotImplementedError("write your pallas kernel here")


# speedup vs baseline: 31.7912x; 31.7912x over previous
import jax
import jax.numpy as jnp
from jax.experimental import pallas as pl
from jax.experimental.pallas import tpu as pltpu

H, S, D = 16, 2048, 64
SINK = 4
NOUT = 32
BQ = 256
NEG = jnp.finfo(jnp.float32).min


def _attn_body(q_ref, k_ref, v_ref, o_ref):
    qi = pl.program_id(1)
    q = q_ref[0]
    k = k_ref[0]
    s = jax.lax.dot_general(q, k, (((1,), (1,)), ((), ())),
                            preferred_element_type=jnp.float32) * (1.0 / 8.0)
    row = qi * BQ + jax.lax.broadcasted_iota(jnp.int32, (BQ, S), 0)
    col = jax.lax.broadcasted_iota(jnp.int32, (BQ, S), 1)
    s = jnp.where(col <= row, s, NEG)
    m = jnp.max(s, axis=1, keepdims=True)
    p = jnp.exp(s - m)
    p = p / jnp.sum(p, axis=1, keepdims=True)
    o_ref[0] = jax.lax.dot_general(p, v_ref[0], (((1,), (0,)), ((), ())),
                                   preferred_element_type=jnp.float32)


def _attention(q, k, v):
    return pl.pallas_call(
        _attn_body,
        grid=(H, S // BQ),
        in_specs=[
            pl.BlockSpec((1, BQ, D), lambda h, i: (h, i, 0)),
            pl.BlockSpec((1, S, D), lambda h, i: (h, 0, 0)),
            pl.BlockSpec((1, S, D), lambda h, i: (h, 0, 0)),
        ],
        out_specs=pl.BlockSpec((1, BQ, D), lambda h, i: (h, i, 0)),
        out_shape=jax.ShapeDtypeStruct((H, S, D), jnp.float32),
    )(q, k, v)


def _topk(sal, n, length, axis):
    shape = (length, 1) if axis == 0 else (1, length)
    iota = jax.lax.broadcasted_iota(jnp.int32, shape, axis)
    slot_row = jax.lax.broadcasted_iota(jnp.int32, (1, n), 1)
    slot_col = jax.lax.broadcasted_iota(jnp.int32, (n, 1), 0)

    def body(i, carry):
        sal, idx_row, idx_col = carry
        m = jnp.max(sal)
        pos = jnp.min(jnp.where(sal == m, iota, length))
        idx_row = jnp.where(slot_row == i, pos, idx_row)
        idx_col = jnp.where(slot_col == i, pos, idx_col)
        sal = jnp.where(iota == pos, -1.0, sal)
        return sal, idx_row, idx_col

    _, idx_row, idx_col = jax.lax.fori_loop(
        0, n, body,
        (sal, jnp.zeros((1, n), jnp.int32), jnp.zeros((n, 1), jnp.int32)))
    return idx_row, idx_col


def _kv1_body(k_ref, v_ref, kr_ref, kc_ref, vr_ref, vc_ref, ksal, vsal):
    h = pl.program_id(0)
    rowi = jax.lax.broadcasted_iota(jnp.int32, (S, D), 0)
    khz = jnp.where(rowi < SINK, 0.0, k_ref[0])
    vhz = jnp.where(rowi < SINK, 0.0, v_ref[0])
    ks = jnp.max(jnp.abs(khz), axis=1, keepdims=True)
    vs = jnp.max(jnp.abs(vhz), axis=0, keepdims=True)

    @pl.when(h == 0)
    def _():
        ksal[...] = ks
        vsal[...] = vs

    @pl.when(h > 0)
    def _():
        ksal[...] = jnp.maximum(ksal[...], ks)
        vsal[...] = jnp.maximum(vsal[...], vs)

    @pl.when(h == H - 1)
    def _():
        kr, kc = _topk(ksal[...], NOUT, S, axis=0)
        kr_ref[...] = kr
        kc_ref[...] = kc
        vr, vc = _topk(vsal[...], NOUT, D, axis=1)
        vr_ref[...] = vr
        vc_ref[...] = vc


def _kv1(k, v):
    return pl.pallas_call(
        _kv1_body,
        grid=(H,),
        in_specs=[
            pl.BlockSpec((1, S, D), lambda h: (h, 0, 0)),
            pl.BlockSpec((1, S, D), lambda h: (h, 0, 0)),
        ],
        out_specs=[
            pl.BlockSpec((1, NOUT), lambda h: (0, 0)),
            pl.BlockSpec((NOUT, 1), lambda h: (0, 0)),
            pl.BlockSpec((1, NOUT), lambda h: (0, 0)),
            pl.BlockSpec((NOUT, 1), lambda h: (0, 0)),
        ],
        out_shape=(
            jax.ShapeDtypeStruct((1, NOUT), jnp.int32),
            jax.ShapeDtypeStruct((NOUT, 1), jnp.int32),
            jax.ShapeDtypeStruct((1, NOUT), jnp.int32),
            jax.ShapeDtypeStruct((NOUT, 1), jnp.int32),
        ),
        scratch_shapes=[
            pltpu.VMEM((S, 1), jnp.float32),
            pltpu.VMEM((1, D), jnp.float32),
        ],
    )(k, v)


def _kv2_body(k_ref, v_ref, kr_ref, kc_ref, vr_ref, vc_ref,
              kq_ref, kscale_ref, ksp_ref, kfi_ref, ksink_ref,
              vq_ref, vscale_ref, vsp_ref, vfi_ref, vsink_ref):
    h = pl.program_id(0)
    rowi = jax.lax.broadcasted_iota(jnp.int32, (S, D), 0)

    kr = kr_ref[...]
    kc = kc_ref[...]
    khz = jnp.where(rowi < SINK, 0.0, k_ref[0])
    cmp_s = (jax.lax.broadcasted_iota(jnp.int32, (S, NOUT), 0) == kr)
    keep = 1.0 - jnp.max(cmp_s.astype(jnp.float32), axis=1, keepdims=True)
    dense = khz * keep
    absmax = jnp.max(jnp.abs(dense), axis=0, keepdims=True)
    scale = jnp.maximum(absmax, 1e-8) / 127.0
    kq_ref[0] = jnp.clip(jnp.round(dense / scale), -127.0, 127.0
                         ).astype(jnp.int8)
    kscale_ref[0] = scale
    onehot = (jax.lax.broadcasted_iota(jnp.int32, (NOUT, S), 1)
              == kc).astype(jnp.float32)
    ksp_ref[0] = jax.lax.dot_general(
        onehot, khz, (((1,), (0,)), ((), ())),
        preferred_element_type=jnp.float32)
    kfi_ref[0] = (h * (S * D) + kc * D
                  + jax.lax.broadcasted_iota(jnp.int32, (NOUT, D), 1))
    ksink_ref[0] = k_ref[0, :SINK, :]

    vr = vr_ref[...]
    vc = vc_ref[...]
    vhz = jnp.where(rowi < SINK, 0.0, v_ref[0])
    cmp_d = (jax.lax.broadcasted_iota(jnp.int32, (NOUT, D), 1) == vc)
    dropd = jnp.max(cmp_d.astype(jnp.float32), axis=0, keepdims=True)
    dense_v = vhz * (1.0 - dropd)
    vabs = jnp.max(jnp.abs(dense_v), axis=1, keepdims=True)
    vsc = jnp.maximum(vabs, 1e-8) / 127.0
    vq_ref[0] = jnp.clip(jnp.round(dense_v / vsc), -127.0, 127.0
                         ).astype(jnp.int8)
    vscale_ref[0] = vsc
    vsp_ref[0] = jax.lax.dot_general(
        vhz, cmp_d.astype(jnp.float32), (((1,), (1,)), ((), ())),
        preferred_element_type=jnp.float32)
    vfi_ref[0] = (h * (S * D)
                  + jax.lax.broadcasted_iota(jnp.int32, (S, NOUT), 0) * D
                  + vr)
    vsink_ref[0] = v_ref[0, :SINK, :]


def _kv2(k, v, kr, kc, vr, vc):
    small = pl.BlockSpec(None, lambda h: (0, 0))
    return pl.pallas_call(
        _kv2_body,
        grid=(H,),
        in_specs=[
            pl.BlockSpec((1, S, D), lambda h: (h, 0, 0)),
            pl.BlockSpec((1, S, D), lambda h: (h, 0, 0)),
            pl.BlockSpec((1, NOUT), lambda h: (0, 0)),
            pl.BlockSpec((NOUT, 1), lambda h: (0, 0)),
            pl.BlockSpec((1, NOUT), lambda h: (0, 0)),
            pl.BlockSpec((NOUT, 1), lambda h: (0, 0)),
        ],
        out_specs=[
            pl.BlockSpec((1, S, D), lambda h: (h, 0, 0)),
            pl.BlockSpec((1, 1, D), lambda h: (h, 0, 0)),
            pl.BlockSpec((1, NOUT, D), lambda h: (h, 0, 0)),
            pl.BlockSpec((1, NOUT, D), lambda h: (h, 0, 0)),
            pl.BlockSpec((1, SINK, D), lambda h: (h, 0, 0)),
            pl.BlockSpec((1, S, D), lambda h: (h, 0, 0)),
            pl.BlockSpec((1, S, 1), lambda h: (h, 0, 0)),
            pl.BlockSpec((1, S, NOUT), lambda h: (h, 0, 0)),
            pl.BlockSpec((1, S, NOUT), lambda h: (h, 0, 0)),
            pl.BlockSpec((1, SINK, D), lambda h: (h, 0, 0)),
        ],
        out_shape=(
            jax.ShapeDtypeStruct((H, S, D), jnp.int8),
            jax.ShapeDtypeStruct((H, 1, D), jnp.float32),
            jax.ShapeDtypeStruct((H, NOUT, D), jnp.float32),
            jax.ShapeDtypeStruct((H, NOUT, D), jnp.int32),
            jax.ShapeDtypeStruct((H, SINK, D), jnp.float32),
            jax.ShapeDtypeStruct((H, S, D), jnp.int8),
            jax.ShapeDtypeStruct((H, S, 1), jnp.float32),
            jax.ShapeDtypeStruct((H, S, NOUT), jnp.float32),
            jax.ShapeDtypeStruct((H, S, NOUT), jnp.int32),
            jax.ShapeDtypeStruct((H, SINK, D), jnp.float32),
        ),
    )(k, v, kr, kc, vr, vc)


def _sink_flat_idx():
    h = jnp.arange(H, dtype=jnp.int32)[:, None, None] * (S * D)
    s = jnp.arange(SINK, dtype=jnp.int32)[None, :, None] * D
    d = jnp.arange(D, dtype=jnp.int32)[None, None, :]
    return (h + s + d).reshape(-1)


def kernel(q_tensor, k_tensor, v_tensor):
    q = q_tensor.reshape(H, S, D)
    k = k_tensor.reshape(H, S, D)
    v = v_tensor.reshape(H, S, D)

    attn = _attention(q, k, v).reshape(1, H, S, D)
    kr, kc, vr, vc = _kv1(k, v)
    (kq, kscale, ksp, kfi, ksink,
     vq, vscale, vsp, vfi, vsink) = _kv2(k, v, kr, kc, vr, vc)

    sink_idx = _sink_flat_idx()
    k_sp_val = jnp.concatenate([ksp.reshape(-1), ksink.reshape(-1)])
    k_sp_idx = jnp.concatenate([kfi.reshape(-1), sink_idx])
    v_sp_val = jnp.concatenate([vsp.reshape(-1), vsink.reshape(-1)])
    v_sp_idx = jnp.concatenate([vfi.reshape(-1), sink_idx])

    return (attn,
            kq.reshape(1, H, S, D),
            kscale.reshape(1, H, 1, D),
            k_sp_val, k_sp_idx,
            vq.reshape(1, H, S, D),
            vscale.reshape(1, H, S, 1),
            v_sp_val, v_sp_idx)
